# Initial kernel scaffold; baseline (speedup 1.0000x reference)
import jax
import jax.numpy as jnp
from jax.experimental import pallas as pl


def kernel(x, lower_index, lower_values, upper_index, upper_values, features_batch, W_low, W_up, W_har, b):
    raise NotImplementedError("write your pallas kernel here")


# speedup vs baseline: 1.2614x; 1.2614x over previous
import functools

import jax
import jax.numpy as jnp
from jax import lax
from jax.experimental import pallas as pl
from jax.experimental.pallas import tpu as pltpu
from jax.experimental.pallas import tpu_sc as plsc

N_ROWS = 10000
ROWS_PER_TILE = 320
NUM_WORKERS = 32
N_PAD = ROWS_PER_TILE * NUM_WORKERS
LANES = 16


def _make_spmm(width, chunk):
    nvec = width // LANES
    mesh = plsc.VectorSubcoreMesh(core_axis_name="c", subcore_axis_name="s")

    @functools.partial(
        pl.kernel,
        out_type=jax.ShapeDtypeStruct((N_PAD, width), jnp.float32),
        mesh=mesh,
        scratch_types=[
            pltpu.VMEM((chunk,), jnp.int32),
            pltpu.VMEM((chunk, width), jnp.float32),
            pltpu.VMEM((ROWS_PER_TILE, width), jnp.float32),
            pltpu.VMEM((chunk,), jnp.float32),
            pltpu.VMEM((chunk,), jnp.int32),
            pltpu.VMEM((48,), jnp.int32),
            pltpu.SemaphoreType.DMA,
        ],
    )
    def spmm(h_hbm, src_hbm, dst_hbm, val_hbm, starts_hbm, out_hbm,
             idx_v, rows_v, acc_v, vals_v, dsts_v, bnds_v, gsem):
        wid = lax.axis_index("c") * 16 + lax.axis_index("s")
        row_base = wid * ROWS_PER_TILE

        pltpu.sync_copy(starts_hbm, bnds_v)
        b0 = bnds_v[pl.ds(0, LANES)]
        b1 = bnds_v[pl.ds(LANES, LANES)]
        b2 = bnds_v[pl.ds(2 * LANES, LANES)]
        ents = ([b0[l] for l in range(LANES)]
                + [b1[l] for l in range(LANES)]
                + [b2[l] for l in range(2)])

        def pick(i):
            r = ents[0]
            for k in range(1, 34):
                r = jnp.where(i == k, ents[k], r)
            return r

        e0 = pick(wid)
        e1 = pick(wid + 1)
        e0a = (e0 // 8) * 8
        nchunks = (e1 - e0a + chunk - 1) // chunk

        def zero_body(r, carry):
            for j in range(nvec):
                acc_v[r, pl.ds(LANES * j, LANES)] = jnp.zeros(
                    (LANES,), jnp.float32)
            return carry
        lax.fori_loop(0, ROWS_PER_TILE, zero_body, 0)

        def chunk_body(ci, carry):
            off = e0a + ci * chunk
            pltpu.sync_copy(src_hbm.at[pl.ds(off, chunk)], idx_v)
            pltpu.sync_copy(val_hbm.at[pl.ds(off, chunk)], vals_v)
            pltpu.sync_copy(dst_hbm.at[pl.ds(off, chunk)], dsts_v)
            pltpu.async_copy(h_hbm.at[idx_v], rows_v, gsem).wait()

            def group_body(grp, c2):
                gbase = grp * LANES
                vv = vals_v[pl.ds(gbase, LANES)]
                dv = dsts_v[pl.ds(gbase, LANES)]
                for l in range(LANES):
                    e = off + gbase + l
                    ok = jnp.logical_and(e >= e0, e < e1)
                    val = jnp.where(ok, vv[l], jnp.float32(0.0))
                    dl = lax.clamp(0, dv[l] - row_base, ROWS_PER_TILE - 1)
                    r = gbase + l
                    for j in range(nvec):
                        g = rows_v[r, pl.ds(LANES * j, LANES)]
                        plsc.addupdate(
                            acc_v.at[dl, pl.ds(LANES * j, LANES)], g * val)
                return c2
            lax.fori_loop(0, chunk // LANES, group_body, 0)
            return carry
        lax.fori_loop(0, nchunks, chunk_body, 0)

        pltpu.sync_copy(acc_v, out_hbm.at[pl.ds(row_base, ROWS_PER_TILE)])

    return spmm


_SPMM_CACHE = {}


def _get_spmm(width):
    if width not in _SPMM_CACHE:
        _SPMM_CACHE[width] = _make_spmm(width, 128 if width == 128 else 64)
    return _SPMM_CACHE[width]


def _dense_layer(h, p1, p2, q1, q2, wh, wl0, wl1, wu0, wu1, bias):
    din = h.shape[1]
    dout = wh.shape[1]
    blk = 512
    grid = (N_PAD // blk,)

    def body(h_ref, p1_ref, p2_ref, q1_ref, q2_ref,
             wh_ref, wl0_ref, wl1_ref, wu0_ref, wu1_ref, b_ref, o_ref):
        acc = jnp.dot(h_ref[...], wh_ref[...],
                      preferred_element_type=jnp.float32)
        acc = acc + jnp.dot(p1_ref[...], wl0_ref[...],
                            preferred_element_type=jnp.float32)
        acc = acc + jnp.dot(p2_ref[...], wl1_ref[...],
                            preferred_element_type=jnp.float32)
        acc = acc + jnp.dot(q1_ref[...], wu0_ref[...],
                            preferred_element_type=jnp.float32)
        acc = acc + jnp.dot(q2_ref[...], wu1_ref[...],
                            preferred_element_type=jnp.float32)
        o_ref[...] = jnp.tanh(acc + b_ref[...])

    row_spec = pl.BlockSpec((blk, din), lambda i: (i, 0))
    w_spec = pl.BlockSpec((din, dout), lambda i: (0, 0))
    return pl.pallas_call(
        body,
        grid=grid,
        in_specs=[row_spec] * 5 + [w_spec] * 5
        + [pl.BlockSpec((1, dout), lambda i: (0, 0))],
        out_specs=pl.BlockSpec((blk, dout), lambda i: (i, 0)),
        out_shape=jax.ShapeDtypeStruct((N_PAD, dout), jnp.float32),
    )(h, p1, p2, q1, q2, wh, wl0, wl1, wu0, wu1, bias.reshape(1, dout))


def _segment_max(h, seg, nseg):
    dout = h.shape[1]
    blk = 512
    grid = (N_PAD // blk,)

    def body(h_ref, seg_ref, o_ref):
        @pl.when(pl.program_id(0) == 0)
        def _():
            o_ref[...] = jnp.full((nseg, dout), -jnp.inf, jnp.float32)
        hv = h_ref[...]
        sv = seg_ref[...]
        for s in range(nseg):
            m = jnp.where(sv == s, hv, -jnp.inf)
            o_ref[s:s + 1, :] = jnp.maximum(
                o_ref[s:s + 1, :], jnp.max(m, axis=0, keepdims=True))

    return pl.pallas_call(
        body,
        grid=grid,
        in_specs=[pl.BlockSpec((blk, dout), lambda i: (i, 0)),
                  pl.BlockSpec((blk, 1), lambda i: (i, 0))],
        out_specs=pl.BlockSpec((nseg, dout), lambda i: (0, 0)),
        out_shape=jax.ShapeDtypeStruct((nseg, dout), jnp.float32),
    )(h, seg.reshape(-1, 1))


def _prep_edges(index, values, max_chunk):
    dst = index[0].astype(jnp.int32)
    src = index[1].astype(jnp.int32)
    order = jnp.argsort(dst)
    dst_s = dst[order]
    src_s = src[order]
    val_s = values[order].astype(jnp.float32)
    bounds = (jnp.arange(33, dtype=jnp.int32) * ROWS_PER_TILE)
    starts = jnp.searchsorted(dst_s, bounds).astype(jnp.int32)
    starts = jnp.concatenate(
        [starts, jnp.full((15,), dst.shape[0], jnp.int32)])
    pad = 2 * max_chunk
    src_p = jnp.concatenate([src_s, jnp.zeros((pad,), jnp.int32)])
    dst_p = jnp.concatenate([dst_s, jnp.zeros((pad,), jnp.int32)])
    val_p = jnp.concatenate([val_s, jnp.zeros((pad,), jnp.float32)])
    return src_p, dst_p, val_p, starts


def kernel(x, lower_index, lower_values, upper_index, upper_values,
           features_batch, W_low, W_up, W_har, b):
    nseg = 16
    kappa = W_low[0].shape[0]
    lsrc, ldst, lval, lstarts = _prep_edges(lower_index, lower_values, 128)
    usrc, udst, uval, ustarts = _prep_edges(upper_index, upper_values, 128)

    h = jnp.zeros((N_PAD, x.shape[1]), jnp.float32).at[:N_ROWS].set(x)
    seg = jnp.full((N_PAD,), nseg, jnp.int32).at[:N_ROWS].set(
        features_batch.astype(jnp.int32))

    for i in range(len(W_har)):
        width = h.shape[1]
        spmm = _get_spmm(width)
        ps = []
        p = h
        for _ in range(kappa):
            p = spmm(p, lsrc, ldst, lval, lstarts)
            ps.append(p)
        q = h
        for _ in range(kappa):
            q = spmm(q, usrc, udst, uval, ustarts)
            ps.append(q)
        h = _dense_layer(h, ps[0], ps[1], ps[2], ps[3],
                         W_har[i], W_low[i][0], W_low[i][1],
                         W_up[i][0], W_up[i][1], b[i])

    return _segment_max(h, seg, nseg)


# speedup vs baseline: 1.7541x; 1.3906x over previous
import functools

import jax
import jax.numpy as jnp
from jax import lax
from jax.experimental import pallas as pl
from jax.experimental.pallas import tpu as pltpu
from jax.experimental.pallas import tpu_sc as plsc

N_ROWS = 10000
ROWS_PER_TILE = 320
NUM_WORKERS = 32
N_PAD = ROWS_PER_TILE * NUM_WORKERS
LANES = 16


def _make_spmm(width, chunk, sub):
    nvec = width // LANES
    sup = chunk * sub
    mesh = plsc.VectorSubcoreMesh(core_axis_name="c", subcore_axis_name="s")

    @functools.partial(
        pl.kernel,
        out_type=jax.ShapeDtypeStruct((N_PAD, width), jnp.float32),
        mesh=mesh,
        scratch_types=[
            pltpu.VMEM((2 * sup,), jnp.int32),
            pltpu.VMEM((2 * sup,), jnp.float32),
            pltpu.VMEM((2 * sup,), jnp.int32),
            pltpu.VMEM((2, chunk, width), jnp.float32),
            pltpu.VMEM((ROWS_PER_TILE, width), jnp.float32),
            pltpu.VMEM((48,), jnp.int32),
            pltpu.SemaphoreType.DMA,
            pltpu.SemaphoreType.DMA((2,)),
        ],
    )
    def spmm(h_hbm, src_hbm, dst_hbm, val_hbm, starts_hbm, out_hbm,
             idx_v, vals_v, dsts_v, rows_v, acc_v, bnds_v, msem, gsem):
        wid = lax.axis_index("c") * 16 + lax.axis_index("s")
        row_base = wid * ROWS_PER_TILE

        pltpu.sync_copy(starts_hbm, bnds_v)
        b0 = bnds_v[pl.ds(0, LANES)]
        b1 = bnds_v[pl.ds(LANES, LANES)]
        b2 = bnds_v[pl.ds(2 * LANES, LANES)]
        ents = ([b0[l] for l in range(LANES)]
                + [b1[l] for l in range(LANES)]
                + [b2[l] for l in range(2)])

        def pick(i):
            r = ents[0]
            for k in range(1, 34):
                r = jnp.where(i == k, ents[k], r)
            return r

        e0 = pick(wid)
        e1 = pick(wid + 1)
        e0a = (e0 // 8) * 8
        nsup = (e1 - e0a + sup - 1) // sup

        def start_meta(si):
            pp = (si % 2) * sup
            off = e0a + si * sup
            pltpu.async_copy(src_hbm.at[pl.ds(off, sup)],
                             idx_v.at[pl.ds(pp, sup)], msem)
            pltpu.async_copy(val_hbm.at[pl.ds(off, sup)],
                             vals_v.at[pl.ds(pp, sup)], msem)
            pltpu.async_copy(dst_hbm.at[pl.ds(off, sup)],
                             dsts_v.at[pl.ds(pp, sup)], msem)

        def wait_meta():
            pltpu.make_async_copy(
                src_hbm.at[pl.ds(0, sup)], idx_v.at[pl.ds(0, sup)],
                msem).wait()
            pltpu.make_async_copy(
                val_hbm.at[pl.ds(0, sup)], vals_v.at[pl.ds(0, sup)],
                msem).wait()
            pltpu.make_async_copy(
                dst_hbm.at[pl.ds(0, sup)], dsts_v.at[pl.ds(0, sup)],
                msem).wait()

        def start_gather(pp, j, parity):
            pltpu.async_copy(
                h_hbm.at[idx_v.at[pl.ds(pp + j * chunk, chunk)]],
                rows_v.at[parity], gsem.at[parity])

        def wait_gather(parity):
            pltpu.make_async_copy(
                h_hbm.at[idx_v.at[pl.ds(0, chunk)]],
                rows_v.at[parity], gsem.at[parity]).wait()

        def zero_body(r, carry):
            for j in range(nvec):
                acc_v[r, pl.ds(LANES * j, LANES)] = jnp.zeros(
                    (LANES,), jnp.float32)
            return carry
        lax.fori_loop(0, ROWS_PER_TILE, zero_body, 0)

        def compute(pp, j, parity, off):
            def group_body(grp, c2):
                gbase = j * chunk + grp * LANES
                vv = vals_v[pl.ds(pp + gbase, LANES)]
                dv = dsts_v[pl.ds(pp + gbase, LANES)]
                for l in range(LANES):
                    e = off + gbase + l
                    ok = jnp.logical_and(e >= e0, e < e1)
                    val = jnp.where(ok, vv[l], jnp.float32(0.0))
                    dl = lax.clamp(0, dv[l] - row_base, ROWS_PER_TILE - 1)
                    r = grp * LANES + l
                    for k in range(nvec):
                        g = rows_v[parity, r, pl.ds(LANES * k, LANES)]
                        plsc.addupdate(
                            acc_v.at[dl, pl.ds(LANES * k, LANES)], g * val)
                return c2
            lax.fori_loop(0, chunk // LANES, group_body, 0)

        start_meta(0)
        wait_meta()
        start_meta(1)
        start_gather(0, 0, 0)

        def super_body(si, carry):
            pp = (si % 2) * sup
            qq = sup - pp

            def pair_body(jp, c):
                for half in range(2):
                    j = jp * 2 + half
                    nxt = 1 - half

                    @pl.when(j < sub - 1)
                    def _():
                        start_gather(pp, j + 1, nxt)

                    @pl.when(j == sub - 1)
                    def _():
                        wait_meta()
                        start_gather(qq, 0, nxt)

                    wait_gather(half)
                    compute(pp, j, half, e0a + si * sup)
                return c
            lax.fori_loop(0, sub // 2, pair_body, 0)
            start_meta(si + 2)
            return carry
        lax.fori_loop(0, nsup, super_body, 0)

        wait_gather(0)
        wait_meta()
        pltpu.sync_copy(acc_v, out_hbm.at[pl.ds(row_base, ROWS_PER_TILE)])

    return spmm


_SPMM_CACHE = {}


def _get_spmm(width):
    if width not in _SPMM_CACHE:
        if width == 128:
            _SPMM_CACHE[width] = _make_spmm(width, 128, 4)
        else:
            _SPMM_CACHE[width] = _make_spmm(width, 64, 8)
    return _SPMM_CACHE[width]


def _dense_layer(h, p1, p2, q1, q2, wh, wl0, wl1, wu0, wu1, bias):
    din = h.shape[1]
    dout = wh.shape[1]
    blk = 512
    grid = (N_PAD // blk,)

    def body(h_ref, p1_ref, p2_ref, q1_ref, q2_ref,
             wh_ref, wl0_ref, wl1_ref, wu0_ref, wu1_ref, b_ref, o_ref):
        acc = jnp.dot(h_ref[...], wh_ref[...],
                      preferred_element_type=jnp.float32)
        acc = acc + jnp.dot(p1_ref[...], wl0_ref[...],
                            preferred_element_type=jnp.float32)
        acc = acc + jnp.dot(p2_ref[...], wl1_ref[...],
                            preferred_element_type=jnp.float32)
        acc = acc + jnp.dot(q1_ref[...], wu0_ref[...],
                            preferred_element_type=jnp.float32)
        acc = acc + jnp.dot(q2_ref[...], wu1_ref[...],
                            preferred_element_type=jnp.float32)
        o_ref[...] = jnp.tanh(acc + b_ref[...])

    row_spec = pl.BlockSpec((blk, din), lambda i: (i, 0))
    w_spec = pl.BlockSpec((din, dout), lambda i: (0, 0))
    return pl.pallas_call(
        body,
        grid=grid,
        in_specs=[row_spec] * 5 + [w_spec] * 5
        + [pl.BlockSpec((1, dout), lambda i: (0, 0))],
        out_specs=pl.BlockSpec((blk, dout), lambda i: (i, 0)),
        out_shape=jax.ShapeDtypeStruct((N_PAD, dout), jnp.float32),
    )(h, p1, p2, q1, q2, wh, wl0, wl1, wu0, wu1, bias.reshape(1, dout))


def _segment_max(h, seg, nseg):
    dout = h.shape[1]
    blk = 512
    grid = (N_PAD // blk,)

    def body(h_ref, seg_ref, o_ref):
        @pl.when(pl.program_id(0) == 0)
        def _():
            o_ref[...] = jnp.full((nseg, dout), -jnp.inf, jnp.float32)
        hv = h_ref[...]
        sv = seg_ref[...]
        for s in range(nseg):
            m = jnp.where(sv == s, hv, -jnp.inf)
            o_ref[s:s + 1, :] = jnp.maximum(
                o_ref[s:s + 1, :], jnp.max(m, axis=0, keepdims=True))

    return pl.pallas_call(
        body,
        grid=grid,
        in_specs=[pl.BlockSpec((blk, dout), lambda i: (i, 0)),
                  pl.BlockSpec((blk, 1), lambda i: (i, 0))],
        out_specs=pl.BlockSpec((nseg, dout), lambda i: (0, 0)),
        out_shape=jax.ShapeDtypeStruct((nseg, dout), jnp.float32),
    )(h, seg.reshape(-1, 1))


def _prep_edges(index, values, pad):
    dst = index[0].astype(jnp.int32)
    src = index[1].astype(jnp.int32)
    order = jnp.argsort(dst)
    dst_s = dst[order]
    src_s = src[order]
    val_s = values[order].astype(jnp.float32)
    bounds = (jnp.arange(33, dtype=jnp.int32) * ROWS_PER_TILE)
    starts = jnp.searchsorted(dst_s, bounds).astype(jnp.int32)
    starts = jnp.concatenate(
        [starts, jnp.full((15,), dst.shape[0], jnp.int32)])
    src_p = jnp.concatenate([src_s, jnp.zeros((pad,), jnp.int32)])
    dst_p = jnp.concatenate([dst_s, jnp.zeros((pad,), jnp.int32)])
    val_p = jnp.concatenate([val_s, jnp.zeros((pad,), jnp.float32)])
    return src_p, dst_p, val_p, starts


def kernel(x, lower_index, lower_values, upper_index, upper_values,
           features_batch, W_low, W_up, W_har, b):
    nseg = 16
    kappa = W_low[0].shape[0]
    lsrc, ldst, lval, lstarts = _prep_edges(lower_index, lower_values, 1600)
    usrc, udst, uval, ustarts = _prep_edges(upper_index, upper_values, 1600)

    h = jnp.zeros((N_PAD, x.shape[1]), jnp.float32).at[:N_ROWS].set(x)
    seg = jnp.full((N_PAD,), nseg, jnp.int32).at[:N_ROWS].set(
        features_batch.astype(jnp.int32))

    for i in range(len(W_har)):
        width = h.shape[1]
        spmm = _get_spmm(width)
        ps = []
        p = h
        for _ in range(kappa):
            p = spmm(p, lsrc, ldst, lval, lstarts)
            ps.append(p)
        q = h
        for _ in range(kappa):
            q = spmm(q, usrc, udst, uval, ustarts)
            ps.append(q)
        h = _dense_layer(h, ps[0], ps[1], ps[2], ps[3],
                         W_har[i], W_low[i][0], W_low[i][1],
                         W_up[i][0], W_up[i][1], b[i])

    return _segment_max(h, seg, nseg)
